# baseline (speedup 1.0000x reference)
import functools

import jax
import jax.numpy as jnp
from jax import lax
from jax.experimental import pallas as pl
from jax.experimental.pallas import tpu as pltpu
from jax.experimental.pallas import tpu_sc as plsc

_SCALE = 1.8
_L = 16
_NC = 2
_NS = 16
_NW = _NC * _NS
_K = 128
_IB = 4
_BLK = 1000


def _sc_mesh():
    return plsc.VectorSubcoreMesh(
        core_axis_name="c", subcore_axis_name="s",
        num_cores=_NC, num_subcores=_NS)


def _worker_base(w, e, ch):
    chk = ch * _K
    raw = w * chk
    clamped = jnp.minimum(raw, e - chk)
    jlo = (raw - clamped) // _K
    return clamped, jlo


@functools.lru_cache(maxsize=None)
def _make_degree_fn(n_pad, ch, e):
    zrows = n_pad // _NS

    @functools.partial(
        pl.kernel,
        out_type=jax.ShapeDtypeStruct((_NC * n_pad,), jnp.float32),
        mesh=_sc_mesh(),
        scratch_types=[
            pltpu.VMEM((ch * _K,), jnp.int32),
            pltpu.VMEM((_K,), jnp.float32),
            pltpu.VMEM((zrows,), jnp.float32),
            pltpu.VMEM_SHARED((n_pad,), jnp.float32),
            pltpu.SemaphoreType.DMA,
        ],
    )
    def deg_fn(dst_hbm, out_hbm, idx_v, ones_v, zero_v, hist_sp, sem):
        del sem
        c = lax.axis_index("c")
        s = lax.axis_index("s")
        w = s * _NC + c
        base, jlo = _worker_base(w, e, ch)
        zero16 = jnp.zeros((_L,), jnp.float32)
        one16 = jnp.ones((_L,), jnp.float32)
        for i in range(zrows // _L):
            zero_v[pl.ds(i * _L, _L)] = zero16
        for i in range(_K // _L):
            ones_v[pl.ds(i * _L, _L)] = one16
        pltpu.sync_copy(zero_v, hist_sp.at[pl.ds(s * zrows, zrows)])
        plsc.subcore_barrier()
        pltpu.sync_copy(dst_hbm.at[pl.ds(base, ch * _K)], idx_v)

        def body(j, carry):
            pltpu.sync_copy(ones_v, hist_sp.at[idx_v.at[pl.ds(j * _K, _K)]],
                            add=True)
            return carry

        lax.fori_loop(jlo, ch, body, 0)
        plsc.subcore_barrier()
        pltpu.sync_copy(hist_sp.at[pl.ds(s * zrows, zrows)],
                        out_hbm.at[pl.ds(c * n_pad + s * zrows, zrows)])

    return deg_fn


@functools.lru_cache(maxsize=None)
def _make_scatter_fn(n_pad, ch, e, d):
    zrows = n_pad // _NS
    nb = ch // _IB
    zr = 8

    @functools.partial(
        pl.kernel,
        out_type=jax.ShapeDtypeStruct((_NC, n_pad, d), jnp.float32),
        mesh=_sc_mesh(),
        scratch_types=[
            pltpu.VMEM((ch * _K,), jnp.int32),
            pltpu.VMEM((_IB * _K,), jnp.int32),
            pltpu.VMEM((_K, d), jnp.float32),
            pltpu.VMEM((_K, d), jnp.float32),
            pltpu.VMEM((zr, d), jnp.float32),
            pltpu.VMEM_SHARED((n_pad, d), jnp.float32),
            pltpu.SemaphoreType.DMA,
            pltpu.SemaphoreType.DMA,
        ],
    )
    def scat_fn(g_hbm, src_hbm, dst_hbm, out_hbm,
                src_v, dst_v, buf0, buf1, zbuf, acc_sp, sem0, sem1):
        c = lax.axis_index("c")
        s = lax.axis_index("s")
        w = s * _NC + c
        base, jlo = _worker_base(w, e, ch)
        blo = jlo // _IB
        zero16 = jnp.zeros((_L,), jnp.float32)
        for i in range(zr):
            for j in range(d // _L):
                zbuf[i, pl.ds(j * _L, _L)] = zero16
        pltpu.sync_copy(src_hbm.at[pl.ds(base, ch * _K)], src_v)
        for r in range(zrows // zr):
            pltpu.sync_copy(zbuf, acc_sp.at[pl.ds(s * zrows + r * zr, zr)])
        plsc.subcore_barrier()

        pltpu.async_copy(g_hbm.at[src_v.at[pl.ds(jlo * _K, _K)]], buf0, sem0)

        def blk(b, carry):
            pltpu.sync_copy(dst_hbm.at[pl.ds(base + b * _IB * _K, _IB * _K)],
                            dst_v)
            for t in range(_IB):
                j = b * _IB + t
                cur, csem = (buf0, sem0) if t % 2 == 0 else (buf1, sem1)
                nxt, nsem = (buf1, sem1) if t % 2 == 0 else (buf0, sem0)

                @pl.when(j + 1 < ch)
                def _():
                    pltpu.async_copy(
                        g_hbm.at[src_v.at[pl.ds((j + 1) * _K, _K)]], nxt, nsem)

                pltpu.make_async_copy(
                    g_hbm.at[src_v.at[pl.ds(j * _K, _K)]], cur, csem).wait()
                pltpu.sync_copy(cur, acc_sp.at[dst_v.at[pl.ds(t * _K, _K)]],
                                add=True)
            return carry

        lax.fori_loop(blo, nb, blk, 0)
        plsc.subcore_barrier()
        pltpu.sync_copy(acc_sp.at[pl.ds(s * zrows, zrows)],
                        out_hbm.at[c, pl.ds(s * zrows, zrows)])

    return scat_fn


def _dense_body(x_ref, w_ref, b_ref, deg_ref, g_ref):
    h = jnp.dot(x_ref[...], w_ref[...], preferred_element_type=jnp.float32)
    h = h + b_ref[...]
    nrm = jnp.sqrt(jnp.sum(h * h, axis=1, keepdims=True))
    scale = lax.rsqrt(deg_ref[...]) * (_SCALE / jnp.maximum(nrm, 1e-12))
    g_ref[...] = h * scale


def _combine_body(tmp_ref, g_ref, deg_ref, o_ref):
    t = tmp_ref[0] + tmp_ref[1] + g_ref[...]
    o_ref[...] = t * lax.rsqrt(deg_ref[...])


def kernel(x, edge_index, W1, b1):
    n, d = x.shape
    e = edge_index.shape[1]
    ch = -(-e // (_NW * _K))
    ch = -(-ch // _IB) * _IB
    assert (_NW * ch * _K - e) % (_IB * _K) == 0
    n_pad = ((n + 255) // 256) * 256

    src = edge_index[0]
    dst = edge_index[1]

    hist = _make_degree_fn(n_pad, ch, e)(dst)
    deg_col = (hist[:n_pad] + hist[n_pad:] + 1.0).reshape(n_pad, 1)

    grid = n // _BLK
    g = pl.pallas_call(
        _dense_body,
        grid=(grid,),
        in_specs=[
            pl.BlockSpec((_BLK, d), lambda i: (i, 0)),
            pl.BlockSpec((d, d), lambda i: (0, 0)),
            pl.BlockSpec((1, d), lambda i: (0, 0)),
            pl.BlockSpec((_BLK, 1), lambda i: (i, 0)),
        ],
        out_specs=pl.BlockSpec((_BLK, d), lambda i: (i, 0)),
        out_shape=jax.ShapeDtypeStruct((n, d), jnp.float32),
    )(x, W1, b1.reshape(1, d), deg_col)

    tmp = _make_scatter_fn(n_pad, ch, e, d)(g, src, dst)

    out = pl.pallas_call(
        _combine_body,
        grid=(grid,),
        in_specs=[
            pl.BlockSpec((_NC, _BLK, d), lambda i: (0, i, 0)),
            pl.BlockSpec((_BLK, d), lambda i: (i, 0)),
            pl.BlockSpec((_BLK, 1), lambda i: (i, 0)),
        ],
        out_specs=pl.BlockSpec((_BLK, d), lambda i: (i, 0)),
        out_shape=jax.ShapeDtypeStruct((n, d), jnp.float32),
    )(tmp, g, deg_col)
    return out


# speedup vs baseline: 1.0859x; 1.0859x over previous
import functools

import jax
import jax.numpy as jnp
from jax import lax
from jax.experimental import pallas as pl
from jax.experimental.pallas import tpu as pltpu
from jax.experimental.pallas import tpu_sc as plsc

_SCALE = 1.8
_L = 16
_NC = 2
_NS = 16
_NW = _NC * _NS
_K = 128
_BLK = 2000


def _sc_mesh():
    return plsc.VectorSubcoreMesh(
        core_axis_name="c", subcore_axis_name="s",
        num_cores=_NC, num_subcores=_NS)


def _worker_base(w, e, ch):
    chk = ch * _K
    raw = w * chk
    clamped = jnp.minimum(raw, e - chk)
    jlo = (raw - clamped) // _K
    return clamped, jlo


@functools.lru_cache(maxsize=None)
def _make_degree_fn(n_pad, ch, e):
    zrows = n_pad // _NS

    @functools.partial(
        pl.kernel,
        out_type=jax.ShapeDtypeStruct((_NC * n_pad,), jnp.float32),
        mesh=_sc_mesh(),
        scratch_types=[
            pltpu.VMEM((2, _K), jnp.int32),
            pltpu.VMEM((2, _K), jnp.int32),
            pltpu.VMEM((_K,), jnp.float32),
            pltpu.VMEM((zrows,), jnp.float32),
            pltpu.VMEM_SHARED((n_pad,), jnp.float32),
            pltpu.SemaphoreType.DMA,
            pltpu.SemaphoreType.DMA,
        ],
    )
    def deg_fn(ei_hbm, out_hbm, ei0, ei1, ones_v, zero_v, hist_sp, si0, si1):
        c = lax.axis_index("c")
        s = lax.axis_index("s")
        w = s * _NC + c
        base, jlo = _worker_base(w, e, ch)
        eis = (ei0, si0), (ei1, si1)

        def ei_src(j):
            return ei_hbm.at[pl.ds(0, 2), pl.ds(base + j * _K, _K)]

        zero16 = jnp.zeros((_L,), jnp.float32)
        one16 = jnp.ones((_L,), jnp.float32)
        for i in range(zrows // _L):
            zero_v[pl.ds(i * _L, _L)] = zero16
        for i in range(_K // _L):
            ones_v[pl.ds(i * _L, _L)] = one16
        pltpu.sync_copy(zero_v, hist_sp.at[pl.ds(s * zrows, zrows)])
        plsc.subcore_barrier()

        pltpu.async_copy(ei_src(jlo), ei0, si0)

        def body(i, carry):
            j = jlo + 2 * i
            for t in range(2):
                cur, csem = eis[t]
                nxt, nsem = eis[1 - t]

                @pl.when(j + t + 1 < ch)
                def _():
                    pltpu.async_copy(ei_src(j + t + 1), nxt, nsem)

                pltpu.make_async_copy(ei_src(j + t), cur, csem).wait()
                pltpu.sync_copy(ones_v, hist_sp.at[cur.at[1]], add=True)
            return carry

        lax.fori_loop(0, (ch - jlo) // 2, body, 0)
        plsc.subcore_barrier()
        pltpu.sync_copy(hist_sp.at[pl.ds(s * zrows, zrows)],
                        out_hbm.at[pl.ds(c * n_pad + s * zrows, zrows)])

    return deg_fn


@functools.lru_cache(maxsize=None)
def _make_scatter_fn(n_pad, ch, e, d):
    zrows = n_pad // _NS
    zr = 8

    @functools.partial(
        pl.kernel,
        out_type=jax.ShapeDtypeStruct((_NC, n_pad, d), jnp.float32),
        mesh=_sc_mesh(),
        scratch_types=[
            [pltpu.VMEM((2, _K), jnp.int32) for _ in range(4)],
            pltpu.VMEM((_K, d), jnp.float32),
            pltpu.VMEM((_K, d), jnp.float32),
            pltpu.VMEM((zr, d), jnp.float32),
            pltpu.VMEM_SHARED((n_pad, d), jnp.float32),
            [pltpu.SemaphoreType.DMA for _ in range(4)],
            pltpu.SemaphoreType.DMA,
            pltpu.SemaphoreType.DMA,
        ],
    )
    def scat_fn(g_hbm, ei_hbm, out_hbm,
                eiv, buf0, buf1, zbuf, acc_sp, sis, sg0, sg1):
        c = lax.axis_index("c")
        s = lax.axis_index("s")
        w = s * _NC + c
        base, jlo = _worker_base(w, e, ch)
        bufs = (buf0, sg0), (buf1, sg1)

        def ei_src(j):
            return ei_hbm.at[pl.ds(0, 2), pl.ds(base + j * _K, _K)]

        zero16 = jnp.zeros((_L,), jnp.float32)
        for i in range(zr):
            for j in range(d // _L):
                zbuf[i, pl.ds(j * _L, _L)] = zero16
        for r in range(zrows // zr):
            pltpu.sync_copy(zbuf, acc_sp.at[pl.ds(s * zrows + r * zr, zr)])
        plsc.subcore_barrier()

        for t in range(4):
            pltpu.async_copy(ei_src(jlo + t), eiv[t], sis[t])
        pltpu.make_async_copy(ei_src(jlo), eiv[0], sis[0]).wait()
        pltpu.async_copy(g_hbm.at[eiv[0].at[0]], buf0, sg0)

        def body(i, carry):
            j = jlo + 4 * i
            for t in range(4):
                jn = j + t + 1
                cur, csem = bufs[t % 2]
                nxt, nsem = bufs[(t + 1) % 2]
                en, esn = eiv[(t + 1) % 4], sis[(t + 1) % 4]

                @pl.when(jn < ch)
                def _():
                    pltpu.make_async_copy(ei_src(jn), en, esn).wait()
                    pltpu.async_copy(g_hbm.at[en.at[0]], nxt, nsem)

                pltpu.make_async_copy(
                    g_hbm.at[eiv[t].at[0]], cur, csem).wait()
                pltpu.sync_copy(cur, acc_sp.at[eiv[t].at[1]], add=True)

                @pl.when(j + t + 4 < ch)
                def _():
                    pltpu.async_copy(ei_src(j + t + 4), eiv[t], sis[t])
            return carry

        lax.fori_loop(0, (ch - jlo) // 4, body, 0)
        plsc.subcore_barrier()
        pltpu.sync_copy(acc_sp.at[pl.ds(s * zrows, zrows)],
                        out_hbm.at[c, pl.ds(s * zrows, zrows)])

    return scat_fn


def _dense_body(x_ref, w_ref, b_ref, deg_ref, g_ref):
    h = jnp.dot(x_ref[...], w_ref[...], preferred_element_type=jnp.float32)
    h = h + b_ref[...]
    nrm = jnp.sqrt(jnp.sum(h * h, axis=1, keepdims=True))
    scale = lax.rsqrt(deg_ref[...]) * (_SCALE / jnp.maximum(nrm, 1e-12))
    g_ref[...] = h * scale


def _combine_body(tmp_ref, g_ref, deg_ref, o_ref):
    t = tmp_ref[0] + tmp_ref[1] + g_ref[...]
    o_ref[...] = t * lax.rsqrt(deg_ref[...])


def kernel(x, edge_index, W1, b1):
    n, d = x.shape
    e = edge_index.shape[1]
    ch = -(-e // (_NW * _K))
    ch = -(-ch // 4) * 4
    assert e % _K == 0 and (_NW * ch * _K - e) % (4 * _K) == 0
    n_pad = ((n + 255) // 256) * 256

    hist = _make_degree_fn(n_pad, ch, e)(edge_index)
    deg_col = (hist[:n_pad] + hist[n_pad:] + 1.0).reshape(n_pad, 1)

    grid = n // _BLK
    g = pl.pallas_call(
        _dense_body,
        grid=(grid,),
        in_specs=[
            pl.BlockSpec((_BLK, d), lambda i: (i, 0)),
            pl.BlockSpec((d, d), lambda i: (0, 0)),
            pl.BlockSpec((1, d), lambda i: (0, 0)),
            pl.BlockSpec((_BLK, 1), lambda i: (i, 0)),
        ],
        out_specs=pl.BlockSpec((_BLK, d), lambda i: (i, 0)),
        out_shape=jax.ShapeDtypeStruct((n, d), jnp.float32),
    )(x, W1, b1.reshape(1, d), deg_col)

    tmp = _make_scatter_fn(n_pad, ch, e, d)(g, edge_index)

    out = pl.pallas_call(
        _combine_body,
        grid=(grid,),
        in_specs=[
            pl.BlockSpec((_NC, _BLK, d), lambda i: (0, i, 0)),
            pl.BlockSpec((_BLK, d), lambda i: (i, 0)),
            pl.BlockSpec((_BLK, 1), lambda i: (i, 0)),
        ],
        out_specs=pl.BlockSpec((_BLK, d), lambda i: (i, 0)),
        out_shape=jax.ShapeDtypeStruct((n, d), jnp.float32),
    )(tmp, g, deg_col)
    return out


# speedup vs baseline: 1.1856x; 1.0918x over previous
import functools

import jax
import jax.numpy as jnp
from jax import lax
from jax.experimental import pallas as pl
from jax.experimental.pallas import tpu as pltpu
from jax.experimental.pallas import tpu_sc as plsc

_SCALE = 1.8
_L = 16
_NC = 2
_NS = 16
_NW = _NC * _NS
_K = 128
_BLK = 2000


def _sc_mesh():
    return plsc.VectorSubcoreMesh(
        core_axis_name="c", subcore_axis_name="s",
        num_cores=_NC, num_subcores=_NS)


def _worker_span(w, e, ch):
    base = w * ch * _K
    jhi = jnp.minimum(ch, (e - base) // _K)
    return base, jhi


@functools.lru_cache(maxsize=None)
def _make_degree_fn(n_pad, ch, e):
    zrows = n_pad // _NS

    @functools.partial(
        pl.kernel,
        out_type=jax.ShapeDtypeStruct((_NC * n_pad,), jnp.float32),
        mesh=_sc_mesh(),
        scratch_types=[
            [pltpu.VMEM((2, _K), jnp.int32) for _ in range(4)],
            pltpu.VMEM((_K,), jnp.float32),
            pltpu.VMEM((zrows,), jnp.float32),
            pltpu.VMEM_SHARED((n_pad,), jnp.float32),
            [pltpu.SemaphoreType.DMA for _ in range(4)],
            [pltpu.SemaphoreType.DMA for _ in range(4)],
        ],
    )
    def deg_fn(ei_hbm, out_hbm, eiv, ones_v, zero_v, hist_sp, sis, sss):
        c = lax.axis_index("c")
        s = lax.axis_index("s")
        w = s * _NC + c
        base, jhi = _worker_span(w, e, ch)

        def ei_src(j):
            return ei_hbm.at[pl.ds(0, 2), pl.ds(base + j * _K, _K)]

        zero16 = jnp.zeros((_L,), jnp.float32)
        one16 = jnp.ones((_L,), jnp.float32)
        for i in range(zrows // _L):
            zero_v[pl.ds(i * _L, _L)] = zero16
        for i in range(_K // _L):
            ones_v[pl.ds(i * _L, _L)] = one16
        pltpu.sync_copy(zero_v, hist_sp.at[pl.ds(s * zrows, zrows)])
        plsc.subcore_barrier()

        for t in range(4):
            pltpu.async_copy(ei_src(t), eiv[t], sis[t])

        def body(i, carry):
            j = 4 * i
            for t in range(4):

                @pl.when(j + t < jhi)
                def _():
                    pltpu.make_async_copy(ei_src(j + t), eiv[t], sis[t]).wait()

                    @pl.when(j + t >= 4)
                    def _():
                        pltpu.make_async_copy(
                            ones_v, hist_sp.at[eiv[t].at[1]], sss[t]).wait()

                    pltpu.async_copy(
                        ones_v, hist_sp.at[eiv[t].at[1]], sss[t], add=True)

                    @pl.when(j + t + 4 < jhi)
                    def _():
                        pltpu.async_copy(ei_src(j + t + 4), eiv[t], sis[t])
            return carry

        lax.fori_loop(0, (jhi + 3) // 4, body, 0)
        for t in range(4):

            @pl.when(t < jhi)
            def _():
                pltpu.make_async_copy(
                    ones_v, hist_sp.at[eiv[t].at[1]], sss[t]).wait()

        plsc.subcore_barrier()
        pltpu.sync_copy(hist_sp.at[pl.ds(s * zrows, zrows)],
                        out_hbm.at[pl.ds(c * n_pad + s * zrows, zrows)])

    return deg_fn


@functools.lru_cache(maxsize=None)
def _make_scatter_fn(n_pad, ch, e, d):
    zrows = n_pad // _NS
    zr = 8

    @functools.partial(
        pl.kernel,
        out_type=jax.ShapeDtypeStruct((_NC, n_pad, d), jnp.float32),
        mesh=_sc_mesh(),
        scratch_types=[
            [pltpu.VMEM((2, _K), jnp.int32) for _ in range(4)],
            pltpu.VMEM((_K, d), jnp.float32),
            pltpu.VMEM((_K, d), jnp.float32),
            pltpu.VMEM((zr, d), jnp.float32),
            pltpu.VMEM_SHARED((n_pad, d), jnp.float32),
            [pltpu.SemaphoreType.DMA for _ in range(4)],
            pltpu.SemaphoreType.DMA,
            pltpu.SemaphoreType.DMA,
        ],
    )
    def scat_fn(g_hbm, ei_hbm, out_hbm,
                eiv, buf0, buf1, zbuf, acc_sp, sis, sg0, sg1):
        c = lax.axis_index("c")
        s = lax.axis_index("s")
        w = s * _NC + c
        base, jhi = _worker_span(w, e, ch)
        bufs = (buf0, sg0), (buf1, sg1)

        def ei_src(j):
            return ei_hbm.at[pl.ds(0, 2), pl.ds(base + j * _K, _K)]

        zero16 = jnp.zeros((_L,), jnp.float32)
        for i in range(zr):
            for j in range(d // _L):
                zbuf[i, pl.ds(j * _L, _L)] = zero16
        for r in range(zrows // zr):
            pltpu.sync_copy(zbuf, acc_sp.at[pl.ds(s * zrows + r * zr, zr)])
        plsc.subcore_barrier()

        for t in range(4):
            pltpu.async_copy(ei_src(t), eiv[t], sis[t])
        pltpu.make_async_copy(ei_src(0), eiv[0], sis[0]).wait()
        pltpu.async_copy(g_hbm.at[eiv[0].at[0]], buf0, sg0)

        def body(i, carry):
            j = 4 * i
            for t in range(4):
                jn = j + t + 1
                cur, csem = bufs[t % 2]
                nxt, nsem = bufs[(t + 1) % 2]
                en, esn = eiv[(t + 1) % 4], sis[(t + 1) % 4]

                @pl.when(jn < jhi)
                def _():
                    pltpu.make_async_copy(ei_src(jn), en, esn).wait()
                    pltpu.async_copy(g_hbm.at[en.at[0]], nxt, nsem)

                @pl.when(j + t < jhi)
                def _():
                    pltpu.make_async_copy(
                        g_hbm.at[eiv[t].at[0]], cur, csem).wait()
                    pltpu.sync_copy(cur, acc_sp.at[eiv[t].at[1]], add=True)

                    @pl.when(j + t + 4 < jhi)
                    def _():
                        pltpu.async_copy(ei_src(j + t + 4), eiv[t], sis[t])
            return carry

        lax.fori_loop(0, (jhi + 3) // 4, body, 0)
        plsc.subcore_barrier()
        pltpu.sync_copy(acc_sp.at[pl.ds(s * zrows, zrows)],
                        out_hbm.at[c, pl.ds(s * zrows, zrows)])

    return scat_fn


def _matnorm_body(x_ref, w_ref, b_ref, hn_ref):
    h = jnp.dot(x_ref[...], w_ref[...], preferred_element_type=jnp.float32)
    h = h + b_ref[...]
    nrm = jnp.sqrt(jnp.sum(h * h, axis=1, keepdims=True))
    hn_ref[...] = h * (_SCALE / jnp.maximum(nrm, 1e-12))


def _scale_body(hn_ref, deg_ref, g_ref):
    g_ref[...] = hn_ref[...] * lax.rsqrt(deg_ref[...])


def _combine_body(tmp_ref, g_ref, deg_ref, o_ref):
    t = tmp_ref[0] + tmp_ref[1] + g_ref[...]
    o_ref[...] = t * lax.rsqrt(deg_ref[...])


def kernel(x, edge_index, W1, b1):
    n, d = x.shape
    e = edge_index.shape[1]
    ch = -(-e // (_NW * _K))
    assert e % _K == 0 and e - (_NW - 1) * ch * _K >= _K
    n_pad = ((n + 255) // 256) * 256

    hist = _make_degree_fn(n_pad, ch, e)(edge_index)
    deg_col = (hist[:n_pad] + hist[n_pad:] + 1.0).reshape(n_pad, 1)

    grid = n // _BLK
    row_spec = pl.BlockSpec((_BLK, d), lambda i: (i, 0))
    hn = pl.pallas_call(
        _matnorm_body,
        grid=(grid,),
        in_specs=[
            row_spec,
            pl.BlockSpec((d, d), lambda i: (0, 0)),
            pl.BlockSpec((1, d), lambda i: (0, 0)),
        ],
        out_specs=row_spec,
        out_shape=jax.ShapeDtypeStruct((n, d), jnp.float32),
    )(x, W1, b1.reshape(1, d))

    g = pl.pallas_call(
        _scale_body,
        grid=(grid,),
        in_specs=[row_spec, pl.BlockSpec((_BLK, 1), lambda i: (i, 0))],
        out_specs=row_spec,
        out_shape=jax.ShapeDtypeStruct((n, d), jnp.float32),
    )(hn, deg_col)

    tmp = _make_scatter_fn(n_pad, ch, e, d)(g, edge_index)

    out = pl.pallas_call(
        _combine_body,
        grid=(grid,),
        in_specs=[
            pl.BlockSpec((_NC, _BLK, d), lambda i: (0, i, 0)),
            row_spec,
            pl.BlockSpec((_BLK, 1), lambda i: (i, 0)),
        ],
        out_specs=row_spec,
        out_shape=jax.ShapeDtypeStruct((n, d), jnp.float32),
    )(tmp, g, deg_col)
    return out


# speedup vs baseline: 1.2192x; 1.0284x over previous
import functools

import jax
import jax.numpy as jnp
from jax import lax
from jax.experimental import pallas as pl
from jax.experimental.pallas import tpu as pltpu
from jax.experimental.pallas import tpu_sc as plsc

_SCALE = 1.8
_L = 16
_NC = 2
_NS = 16
_NW = _NC * _NS
_K = 128
_BLK = 2000


def _sc_mesh():
    return plsc.VectorSubcoreMesh(
        core_axis_name="c", subcore_axis_name="s",
        num_cores=_NC, num_subcores=_NS)


def _worker_span(w, e, ch):
    base = w * ch * _K
    jhi = jnp.minimum(ch, (e - base) // _K)
    return base, jhi


@functools.lru_cache(maxsize=None)
def _make_degree_fn(n_pad, ch, e):
    zrows = n_pad // _NS

    @functools.partial(
        pl.kernel,
        out_type=jax.ShapeDtypeStruct((_NC * n_pad,), jnp.float32),
        mesh=_sc_mesh(),
        scratch_types=[
            [pltpu.VMEM((2, _K), jnp.int32) for _ in range(4)],
            pltpu.VMEM((_K,), jnp.float32),
            pltpu.VMEM((zrows,), jnp.float32),
            pltpu.VMEM_SHARED((n_pad,), jnp.float32),
            [pltpu.SemaphoreType.DMA for _ in range(4)],
            [pltpu.SemaphoreType.DMA for _ in range(4)],
        ],
    )
    def deg_fn(ei_hbm, out_hbm, eiv, ones_v, zero_v, hist_sp, sis, sss):
        c = lax.axis_index("c")
        s = lax.axis_index("s")
        w = s * _NC + c
        base, jhi = _worker_span(w, e, ch)

        def ei_src(j):
            return ei_hbm.at[pl.ds(0, 2), pl.ds(base + j * _K, _K)]

        zero16 = jnp.zeros((_L,), jnp.float32)
        one16 = jnp.ones((_L,), jnp.float32)
        for i in range(zrows // _L):
            zero_v[pl.ds(i * _L, _L)] = zero16
        for i in range(_K // _L):
            ones_v[pl.ds(i * _L, _L)] = one16
        pltpu.sync_copy(zero_v, hist_sp.at[pl.ds(s * zrows, zrows)])
        plsc.subcore_barrier()

        for t in range(4):
            pltpu.async_copy(ei_src(t), eiv[t], sis[t])

        def body(i, carry):
            j = 4 * i
            for t in range(4):

                @pl.when(j + t < jhi)
                def _():
                    pltpu.make_async_copy(ei_src(j + t), eiv[t], sis[t]).wait()

                    @pl.when(j + t >= 4)
                    def _():
                        pltpu.make_async_copy(
                            ones_v, hist_sp.at[eiv[t].at[1]], sss[t]).wait()

                    pltpu.async_copy(
                        ones_v, hist_sp.at[eiv[t].at[1]], sss[t], add=True)

                    @pl.when(j + t + 4 < jhi)
                    def _():
                        pltpu.async_copy(ei_src(j + t + 4), eiv[t], sis[t])
            return carry

        lax.fori_loop(0, (jhi + 3) // 4, body, 0)
        for t in range(4):

            @pl.when(t < jhi)
            def _():
                pltpu.make_async_copy(
                    ones_v, hist_sp.at[eiv[t].at[1]], sss[t]).wait()

        plsc.subcore_barrier()
        pltpu.sync_copy(hist_sp.at[pl.ds(s * zrows, zrows)],
                        out_hbm.at[pl.ds(c * n_pad + s * zrows, zrows)])

    return deg_fn


@functools.lru_cache(maxsize=None)
def _make_scatter_fn(n_pad, ch, e, d):
    zrows = n_pad // _NS
    zr = 16

    @functools.partial(
        pl.kernel,
        out_type=jax.ShapeDtypeStruct((_NC, n_pad, d), jnp.float32),
        mesh=_sc_mesh(),
        scratch_types=[
            [pltpu.VMEM((2, _K), jnp.int32) for _ in range(4)],
            pltpu.VMEM((_K, d), jnp.float32),
            pltpu.VMEM((_K, d), jnp.float32),
            pltpu.VMEM((zr, d), jnp.float32),
            pltpu.VMEM_SHARED((n_pad, d), jnp.float32),
            [pltpu.SemaphoreType.DMA for _ in range(4)],
            pltpu.SemaphoreType.DMA,
            pltpu.SemaphoreType.DMA,
        ],
    )
    def scat_fn(g_hbm, ei_hbm, out_hbm,
                eiv, buf0, buf1, zbuf, acc_sp, sis, sg0, sg1):
        c = lax.axis_index("c")
        s = lax.axis_index("s")
        w = s * _NC + c
        base, jhi = _worker_span(w, e, ch)
        bufs = (buf0, sg0), (buf1, sg1)

        def ei_src(j):
            return ei_hbm.at[pl.ds(0, 2), pl.ds(base + j * _K, _K)]

        zero16 = jnp.zeros((_L,), jnp.float32)
        for i in range(zr):
            for j in range(d // _L):
                zbuf[i, pl.ds(j * _L, _L)] = zero16
        for r in range(zrows // zr):
            pltpu.async_copy(zbuf, acc_sp.at[pl.ds(s * zrows + r * zr, zr)],
                             sg0 if r % 2 == 0 else sg1)
        for r in range(zrows // zr):
            pltpu.make_async_copy(
                zbuf, acc_sp.at[pl.ds(s * zrows + r * zr, zr)],
                sg0 if r % 2 == 0 else sg1).wait()
        plsc.subcore_barrier()

        for t in range(4):
            pltpu.async_copy(ei_src(t), eiv[t], sis[t])
        pltpu.make_async_copy(ei_src(0), eiv[0], sis[0]).wait()
        pltpu.async_copy(g_hbm.at[eiv[0].at[0]], buf0, sg0)

        def body(i, carry):
            j = 4 * i
            for t in range(4):
                jn = j + t + 1
                cur, csem = bufs[t % 2]
                nxt, nsem = bufs[(t + 1) % 2]
                en, esn = eiv[(t + 1) % 4], sis[(t + 1) % 4]

                @pl.when(jn < jhi)
                def _():
                    pltpu.make_async_copy(ei_src(jn), en, esn).wait()
                    pltpu.async_copy(g_hbm.at[en.at[0]], nxt, nsem)

                @pl.when(j + t < jhi)
                def _():
                    pltpu.make_async_copy(
                        g_hbm.at[eiv[t].at[0]], cur, csem).wait()
                    pltpu.sync_copy(cur, acc_sp.at[eiv[t].at[1]], add=True)

                    @pl.when(j + t + 4 < jhi)
                    def _():
                        pltpu.async_copy(ei_src(j + t + 4), eiv[t], sis[t])
            return carry

        lax.fori_loop(0, (jhi + 3) // 4, body, 0)
        plsc.subcore_barrier()
        pltpu.sync_copy(acc_sp.at[pl.ds(s * zrows, zrows)],
                        out_hbm.at[c, pl.ds(s * zrows, zrows)])

    return scat_fn


def _matnorm_body(x_ref, w_ref, b_ref, hn_ref):
    h = jnp.dot(x_ref[...], w_ref[...], preferred_element_type=jnp.float32)
    h = h + b_ref[...]
    nrm = jnp.sqrt(jnp.sum(h * h, axis=1, keepdims=True))
    hn_ref[...] = h * (_SCALE / jnp.maximum(nrm, 1e-12))


def _scale_body(hn_ref, deg_ref, g_ref):
    g_ref[...] = hn_ref[...] * lax.rsqrt(deg_ref[...])


def _combine_body(tmp_ref, g_ref, deg_ref, o_ref):
    t = tmp_ref[0] + tmp_ref[1] + g_ref[...]
    o_ref[...] = t * lax.rsqrt(deg_ref[...])


def kernel(x, edge_index, W1, b1):
    n, d = x.shape
    e = edge_index.shape[1]
    ch = -(-e // (_NW * _K))
    assert e % _K == 0 and e - (_NW - 1) * ch * _K >= _K
    n_pad = ((n + 255) // 256) * 256

    hist = _make_degree_fn(n_pad, ch, e)(edge_index)
    deg_col = (hist[:n_pad] + hist[n_pad:] + 1.0).reshape(n_pad, 1)

    grid = n // _BLK
    row_spec = pl.BlockSpec((_BLK, d), lambda i: (i, 0))
    hn = pl.pallas_call(
        _matnorm_body,
        grid=(grid,),
        in_specs=[
            row_spec,
            pl.BlockSpec((d, d), lambda i: (0, 0)),
            pl.BlockSpec((1, d), lambda i: (0, 0)),
        ],
        out_specs=row_spec,
        out_shape=jax.ShapeDtypeStruct((n, d), jnp.float32),
    )(x, W1, b1.reshape(1, d))

    g = pl.pallas_call(
        _scale_body,
        grid=(grid,),
        in_specs=[row_spec, pl.BlockSpec((_BLK, 1), lambda i: (i, 0))],
        out_specs=row_spec,
        out_shape=jax.ShapeDtypeStruct((n, d), jnp.float32),
    )(hn, deg_col)

    tmp = _make_scatter_fn(n_pad, ch, e, d)(g, edge_index)

    out = pl.pallas_call(
        _combine_body,
        grid=(grid,),
        in_specs=[
            pl.BlockSpec((_NC, _BLK, d), lambda i: (0, i, 0)),
            row_spec,
            pl.BlockSpec((_BLK, 1), lambda i: (i, 0)),
        ],
        out_specs=row_spec,
        out_shape=jax.ShapeDtypeStruct((n, d), jnp.float32),
    )(tmp, g, deg_col)
    return out
